# Initial kernel scaffold; baseline (speedup 1.0000x reference)
import jax
import jax.numpy as jnp
from jax.experimental import pallas as pl


def kernel(sample_values, ray_ids):
    raise NotImplementedError("write your pallas kernel here")


# speedup vs baseline: 1.6087x; 1.6087x over previous
import functools

import jax
import jax.numpy as jnp
from jax.experimental import pallas as pl
from jax.experimental.pallas import tpu as pltpu

_N = 262144
_R = 4096
_D = 32
_B = 256


def _scan_body(v_ref, ids_ref, out_r_ref, out_s_ref, carry_ref, cid_ref):
    i = pl.program_id(0)
    b = v_ref.shape[0]
    ids = ids_ref[0, 0, :]
    v = v_ref[...]

    @pl.when(i == 0)
    def _init():
        carry_ref[...] = jnp.zeros_like(carry_ref)
        cid_ref[0] = -1
        out_r_ref[...] = jnp.zeros_like(out_r_ref)

    ids_col = ids.reshape(b, 1)
    ids_row = ids.reshape(1, b)
    row_i = jax.lax.broadcasted_iota(jnp.int32, (b, b), 0)
    col_j = jax.lax.broadcasted_iota(jnp.int32, (b, b), 1)
    maskf = ((ids_col == ids_row) & (col_j <= row_i)).astype(jnp.float32)
    ps = jax.lax.dot_general(
        maskf, v, (((1,), (0,)), ((), ())), preferred_element_type=jnp.float32
    )
    carry_match = (ids == cid_ref[0]).astype(jnp.float32).reshape(b, 1)
    ps = ps + carry_match * carry_ref[...]
    out_s_ref[...] = ps

    rr = jax.lax.broadcasted_iota(jnp.int32, (_R, b), 0)
    oh = (rr == ids_row).astype(jnp.float32)
    out_r_ref[...] += jax.lax.dot_general(
        oh, v, (((1,), (0,)), ((), ())), preferred_element_type=jnp.float32
    )

    carry_ref[...] = ps[b - 1 : b, :]
    cid_ref[0] = ids[b - 1]


@jax.jit
def kernel(sample_values, ray_ids):
    n, d = sample_values.shape
    nb = n // _B
    ids3 = ray_ids.astype(jnp.int32).reshape(nb, 1, _B)

    out_ray, out_sample = pl.pallas_call(
        _scan_body,
        grid=(nb,),
        in_specs=[
            pl.BlockSpec((_B, d), lambda i: (i, 0)),
            pl.BlockSpec((1, 1, _B), lambda i: (i, 0, 0)),
        ],
        out_specs=[
            pl.BlockSpec((_R, d), lambda i: (0, 0)),
            pl.BlockSpec((_B, d), lambda i: (i, 0)),
        ],
        out_shape=[
            jax.ShapeDtypeStruct((_R, d), jnp.float32),
            jax.ShapeDtypeStruct((n, d), jnp.float32),
        ],
        scratch_shapes=[
            pltpu.VMEM((1, d), jnp.float32),
            pltpu.SMEM((1,), jnp.int32),
        ],
    )(sample_values, ids3)
    return out_ray, out_sample


# speedup vs baseline: 1.8499x; 1.1500x over previous
import functools

import jax
import jax.numpy as jnp
from jax import lax
from jax.experimental import pallas as pl
from jax.experimental.pallas import tpu as pltpu
from jax.experimental.pallas import tpu_sc as plsc

_N = 262144
_R = 4096
_D = 32
_B = 256

_SC_SUB = 16
_SC_CHUNK = _N // _SC_SUB
_SC_T = 1024
_SC_NT = _SC_CHUNK // _SC_T
_R_SLICE = _R // _SC_SUB


def _scan_body(v_ref, ids_ref, out_s_ref, carry_ref, cid_ref):
    i = pl.program_id(0)
    b = v_ref.shape[0]
    ids = ids_ref[0, 0, :]
    v = v_ref[...]

    @pl.when(i == 0)
    def _init():
        carry_ref[...] = jnp.zeros_like(carry_ref)
        cid_ref[0] = -1

    ids_col = ids.reshape(b, 1)
    ids_row = ids.reshape(1, b)
    row_i = jax.lax.broadcasted_iota(jnp.int32, (b, b), 0)
    col_j = jax.lax.broadcasted_iota(jnp.int32, (b, b), 1)
    maskf = ((ids_col == ids_row) & (col_j <= row_i)).astype(jnp.float32)
    ps = jax.lax.dot_general(
        maskf, v, (((1,), (0,)), ((), ())), preferred_element_type=jnp.float32
    )
    carry_match = (ids == cid_ref[0]).astype(jnp.float32).reshape(b, 1)
    ps = ps + carry_match * carry_ref[...]
    out_s_ref[...] = ps

    carry_ref[...] = ps[b - 1 : b, :]
    cid_ref[0] = ids[b - 1]


def _per_sample_call(sample_values, ids3):
    n, d = sample_values.shape
    nb = n // _B
    return pl.pallas_call(
        _scan_body,
        grid=(nb,),
        in_specs=[
            pl.BlockSpec((_B, d), lambda i: (i, 0)),
            pl.BlockSpec((1, 1, _B), lambda i: (i, 0, 0)),
        ],
        out_specs=pl.BlockSpec((_B, d), lambda i: (i, 0)),
        out_shape=jax.ShapeDtypeStruct((n, d), jnp.float32),
        scratch_shapes=[
            pltpu.VMEM((1, d), jnp.float32),
            pltpu.SMEM((1,), jnp.int32),
        ],
    )(sample_values, ids3)


def _sc_perray_body(v_hbm, ids_hbm, out_hbm, acc, zbuf, rows, idx):
    cid = lax.axis_index("c")
    sid = lax.axis_index("s")

    @pl.when(cid == 0)
    def _core0():
        def _zb(i, c):
            zbuf[i, pl.ds(0, 16)] = jnp.zeros((16,), jnp.float32)
            zbuf[i, pl.ds(16, 16)] = jnp.zeros((16,), jnp.float32)
            return c

        lax.fori_loop(0, _R_SLICE, _zb, 0)
        racc0 = pl.multiple_of(sid * _R_SLICE, 8)
        pltpu.sync_copy(zbuf, acc.at[pl.ds(racc0, _R_SLICE)])
        plsc.subcore_barrier()

        def _tile(t, c):
            off = pl.multiple_of(sid * _SC_CHUNK + t * _SC_T, 8)
            pltpu.sync_copy(v_hbm.at[pl.ds(off, _SC_T)], rows)
            ioff = pl.multiple_of(
                sid * (_SC_CHUNK // 128) + t * (_SC_T // 128), 8
            )
            pltpu.sync_copy(ids_hbm.at[pl.ds(ioff, _SC_T // 128)], idx)
            for j in range(_SC_T // 128):
                pltpu.sync_copy(
                    rows.at[pl.ds(j * 128, 128)], acc.at[idx.at[j]], add=True
                )
            return c

        lax.fori_loop(0, _SC_NT, _tile, 0)

        plsc.subcore_barrier()
        pltpu.sync_copy(
            acc.at[pl.ds(sid * _R_SLICE, _R_SLICE)],
            out_hbm.at[pl.ds(sid * _R_SLICE, _R_SLICE)],
        )


@functools.partial(
    pl.kernel,
    out_type=jax.ShapeDtypeStruct((_R, _D), jnp.float32),
    mesh=plsc.VectorSubcoreMesh(core_axis_name="c", subcore_axis_name="s"),
    compiler_params=pltpu.CompilerParams(use_tc_tiling_on_sc=False),
    scratch_types=[
        pltpu.VMEM_SHARED((_R, _D), jnp.float32),
        pltpu.VMEM((_R_SLICE, _D), jnp.float32),
        pltpu.VMEM((_SC_T, _D), jnp.float32),
        pltpu.VMEM((_SC_T // 128, 128), jnp.int32),
    ],
)
def _per_ray_call(v_hbm, ids_hbm, out_hbm, acc, zbuf, rows, idx):
    _sc_perray_body(v_hbm, ids_hbm, out_hbm, acc, zbuf, rows, idx)


@jax.jit
def kernel(sample_values, ray_ids):
    n, d = sample_values.shape
    nb = n // _B
    ids32 = ray_ids.astype(jnp.int32)
    ids3 = ids32.reshape(nb, 1, _B)
    ids2d = ids32.reshape(n // 128, 128)

    out_ray = _per_ray_call(sample_values, ids2d)
    out_sample = _per_sample_call(sample_values, ids3)
    return out_ray, out_sample
